# baseline (speedup 1.0000x reference)
import functools

import jax
import jax.numpy as jnp
import numpy as np
from jax import lax
from jax.experimental import pallas as pl
from jax.experimental.pallas import tpu as pltpu
from jax.experimental.pallas import tpu_sc as plsc

B = 128
V = 100000
VC = 2048
NSTEPS = (V + VC - 1) // VC

_NEG_INF = np.float32(-np.inf)
_TINY = np.float32(1.1754944e-38)
_BIG_I32 = np.int32(2147483647)

NC = 2
NS = 16
NW = NC * NS
RG = 8
CSPLIT = 49920
CHW = 3840
NCH = 13
CTAIL = V - CSPLIT - NCH * CHW


def _sc_body(logits_hbm, actions_hbm, tail_hbm, m_out, s_out, t_out, mv_out,
             mi_out, la_out, buf, tbuf, a_buf,
             acc_m, acc_s, acc_t, acc_mv, acc_mi, acc_la):
    wid = lax.axis_index("s") * NC + lax.axis_index("c")
    rg = wid // 2
    half = wid % 2
    cbase = half * CSPLIT
    lane = lax.iota(jnp.int32, 16)

    pltpu.sync_copy(actions_hbm, a_buf)

    neg = jnp.full((16,), _NEG_INF, jnp.float32)
    zf = jnp.zeros((16,), jnp.float32)
    zi = jnp.zeros((16,), jnp.int32)
    for i in range(RG):
        acc_m[i, pl.ds(0, 16)] = neg
        acc_s[i, pl.ds(0, 16)] = zf
        acc_t[i, pl.ds(0, 16)] = zf
        acc_mv[i, pl.ds(0, 16)] = neg
        acc_mi[i, pl.ds(0, 16)] = zi
        acc_la[i, pl.ds(0, 16)] = zf

    def consume(src_buf, c0, width, nvr):
        for i in range(RG):
            m = acc_m[i, pl.ds(0, 16)]
            s = acc_s[i, pl.ds(0, 16)]
            t = acc_t[i, pl.ds(0, 16)]
            mv = acc_mv[i, pl.ds(0, 16)]
            mi = acc_mi[i, pl.ds(0, 16)]
            la = acc_la[i, pl.ds(0, 16)]
            av = a_buf[i, pl.ds(pl.multiple_of(rg * 16, 16), 16)]

            def vreg_body(q, c2):
                m, s, t, mv, mi, la = c2
                v = src_buf[i, pl.ds(q * 16, 16)]
                col = c0 + q * 16 + lane
                m_new = jnp.maximum(m, v)
                scale = jnp.exp(m - m_new)
                ev = jnp.exp(v - m_new)
                s = s * scale + ev
                t = t * scale + ev * v
                upd = v > mv
                mv = jnp.where(upd, v, mv)
                mi = jnp.where(upd, col, mi)
                colf = lax.convert_element_type(col, jnp.float32)
                la = jnp.where(colf == av, v, la)
                return (m_new, s, t, mv, mi, la)

            m, s, t, mv, mi, la = lax.fori_loop(
                0, nvr, vreg_body, (m, s, t, mv, mi, la), unroll=4
            )

            acc_m[i, pl.ds(0, 16)] = m
            acc_s[i, pl.ds(0, 16)] = s
            acc_t[i, pl.ds(0, 16)] = t
            acc_mv[i, pl.ds(0, 16)] = mv
            acc_mi[i, pl.ds(0, 16)] = mi
            acc_la[i, pl.ds(0, 16)] = la

    def chunk_body(k, carry):
        c0 = cbase + k * CHW
        pltpu.sync_copy(logits_hbm.at[pl.ds(rg * RG, RG), pl.ds(c0, CHW)], buf)
        consume(buf, c0, CHW, CHW // 16)
        return carry

    lax.fori_loop(0, NCH, chunk_body, 0)

    @pl.when(half == 1)
    def _tail():
        pltpu.sync_copy(tail_hbm.at[pl.ds(rg * RG, RG), :], tbuf)
        consume(tbuf, CSPLIT + NCH * CHW, CTAIL, CTAIL // 16)

    pltpu.sync_copy(acc_m, m_out.at[wid])
    pltpu.sync_copy(acc_s, s_out.at[wid])
    pltpu.sync_copy(acc_t, t_out.at[wid])
    pltpu.sync_copy(acc_mv, mv_out.at[wid])
    pltpu.sync_copy(acc_mi, mi_out.at[wid])
    pltpu.sync_copy(acc_la, la_out.at[wid])


def _sc_reduce(logits, actions):
    f32 = jnp.float32
    out_type = (
        jax.ShapeDtypeStruct((NW, RG, 16), f32),
        jax.ShapeDtypeStruct((NW, RG, 16), f32),
        jax.ShapeDtypeStruct((NW, RG, 16), f32),
        jax.ShapeDtypeStruct((NW, RG, 16), f32),
        jax.ShapeDtypeStruct((NW, RG, 16), jnp.int32),
        jax.ShapeDtypeStruct((NW, RG, 16), f32),
    )
    a_tab = jnp.broadcast_to(
        actions.reshape(16, 8).T[:, :, None], (8, 16, 16)
    ).reshape(8, 256).astype(jnp.float32)
    tail = jax.lax.slice(logits, (0, CSPLIT + NCH * CHW), (B, V))
    mesh = plsc.VectorSubcoreMesh(core_axis_name="c", subcore_axis_name="s")
    fn = pl.kernel(
        _sc_body,
        out_type=out_type,
        mesh=mesh,
        scratch_types=[
            pltpu.VMEM((RG, CHW), f32),
            pltpu.VMEM((RG, CTAIL), f32),
            pltpu.VMEM((RG, 256), jnp.float32),
            pltpu.VMEM((RG, 16), f32),
            pltpu.VMEM((RG, 16), f32),
            pltpu.VMEM((RG, 16), f32),
            pltpu.VMEM((RG, 16), f32),
            pltpu.VMEM((RG, 16), jnp.int32),
            pltpu.VMEM((RG, 16), f32),
        ],
    )
    return fn(logits, a_tab, tail)


def _rotl(x, r):
    return jnp.bitwise_or(
        jnp.left_shift(x, jnp.uint32(r)), jnp.right_shift(x, jnp.uint32(32 - r))
    )


def _threefry_bits(n):
    ks0 = jnp.uint32(0)
    ks1 = jnp.uint32(1)
    ks2 = jnp.uint32(0x1BD11BDA ^ 0 ^ 1)
    x0 = jnp.zeros_like(n) + ks0
    x1 = n + ks1
    rots = ((13, 15, 26, 6), (17, 29, 16, 24))
    ks = (ks0, ks1, ks2)
    for i in range(5):
        for r in rots[i % 2]:
            x0 = x0 + x1
            x1 = _rotl(x1, r)
            x1 = jnp.bitwise_xor(x1, x0)
        x0 = x0 + ks[(i + 1) % 3]
        x1 = x1 + ks[(i + 2) % 3] + jnp.uint32(i + 1)
    return jnp.bitwise_xor(x0, x1)


def _gumbel_from_bits(bits):
    fb = jnp.bitwise_or(jnp.right_shift(bits, jnp.uint32(9)), jnp.uint32(0x3F800000))
    f = jax.lax.bitcast_convert_type(fb, jnp.float32) - jnp.float32(1.0)
    u = jnp.maximum(_TINY, f)
    return -jnp.log(-jnp.log(u))


def _sample_chunk(x, c0, carry):
    sampv, sampi = carry
    w = x.shape[1]
    col = jax.lax.broadcasted_iota(jnp.int32, (B, w), 1) + c0
    n = (jax.lax.broadcasted_iota(jnp.int32, (B, w), 0) * V + col).astype(jnp.uint32)
    g = _gumbel_from_bits(_threefry_bits(n))
    y = x + g
    ymax = jnp.max(y, axis=1, keepdims=True)
    yidx = jnp.min(jnp.where(y == ymax, col, _BIG_I32), axis=1, keepdims=True)
    ybetter = ymax > sampv
    return (jnp.where(ybetter, ymax, sampv), jnp.where(ybetter, yidx, sampi))


NFULL = 48
TAILW = V - NFULL * VC


def _sample_kernel(logits_hbm, sample_out, buf0, buf1, tbuf, sem0, sem1):
    def start(chunk, buf, sem):
        pltpu.make_async_copy(
            logits_hbm.at[:, pl.ds(chunk * VC, VC)], buf, sem
        ).start()

    def wait(chunk, buf, sem):
        pltpu.make_async_copy(
            logits_hbm.at[:, pl.ds(chunk * VC, VC)], buf, sem
        ).wait()

    def start_tail(sem):
        pltpu.make_async_copy(
            logits_hbm.at[:, pl.ds(NFULL * VC, TAILW)], tbuf, sem
        ).start()

    def wait_tail(sem):
        pltpu.make_async_copy(
            logits_hbm.at[:, pl.ds(NFULL * VC, TAILW)], tbuf, sem
        ).wait()

    start(0, buf0, sem0)

    def body(j2, carry):
        ca = 2 * j2
        start(ca + 1, buf1, sem1)
        wait(ca, buf0, sem0)
        carry = _sample_chunk(buf0[...], ca * VC, carry)

        @pl.when(j2 < NFULL // 2 - 1)
        def _nx():
            start(ca + 2, buf0, sem0)

        @pl.when(j2 == NFULL // 2 - 1)
        def _tl():
            start_tail(sem0)

        wait(ca + 1, buf1, sem1)
        carry = _sample_chunk(buf1[...], (ca + 1) * VC, carry)
        return carry

    carry0 = (
        jnp.full((B, 1), _NEG_INF, jnp.float32),
        jnp.zeros((B, 1), jnp.int32),
    )
    sampv, sampi = jax.lax.fori_loop(0, NFULL // 2, body, carry0)

    wait_tail(sem0)
    sampv, sampi = _sample_chunk(tbuf[...], NFULL * VC, (sampv, sampi))
    sample_out[...] = sampi


def _tc_sample(logits):
    return pl.pallas_call(
        _sample_kernel,
        in_specs=[pl.BlockSpec(memory_space=pl.ANY)],
        out_specs=pl.BlockSpec(memory_space=pltpu.MemorySpace.VMEM),
        out_shape=jax.ShapeDtypeStruct((B, 1), jnp.int32),
        scratch_shapes=[
            pltpu.VMEM((B, VC), jnp.float32),
            pltpu.VMEM((B, VC), jnp.float32),
            pltpu.VMEM((B, TAILW), jnp.float32),
            pltpu.SemaphoreType.DMA,
            pltpu.SemaphoreType.DMA,
        ],
    )(logits)


def _rows(o, h):
    return o[h::2].reshape(B, 16)


@functools.partial(jax.jit)
def kernel(logits, actions):
    m, s, t, mv, mi, la = _sc_reduce(logits, actions)
    sample = _tc_sample(logits)

    mm = jnp.concatenate([_rows(m, 0), _rows(m, 1)], axis=1)
    ss = jnp.concatenate([_rows(s, 0), _rows(s, 1)], axis=1)
    tt = jnp.concatenate([_rows(t, 0), _rows(t, 1)], axis=1)
    M = jnp.max(mm, axis=1)
    w = jnp.exp(mm - M[:, None])
    S = jnp.sum(ss * w, axis=1)
    T = jnp.sum(tt * w, axis=1)

    mvv = jnp.concatenate([_rows(mv, 0), _rows(mv, 1)], axis=1)
    mii = jnp.concatenate([_rows(mi, 0), _rows(mi, 1)], axis=1)
    MV = jnp.max(mvv, axis=1, keepdims=True)
    MI = jnp.min(jnp.where(mvv == MV, mii, _BIG_I32), axis=1)

    LA = jnp.sum(_rows(la, 0) + _rows(la, 1), axis=1)

    lse = M + jnp.log(S)
    lp = (LA - lse)[:, None]
    ent = lse - T / S
    mode = MI[:, None]
    return (lp, ent, mode, sample)
